# baseline (speedup 1.0000x reference)
import functools

import jax
import jax.numpy as jnp
from jax import lax
from jax.experimental import pallas as pl
from jax.experimental.pallas import tpu as pltpu
from jax.experimental.pallas import tpu_sc as plsc

NU = 4096
NI = 4096
D = 64
IF_LR = 0.01
NC = 2
NS = 16
L = 16
NW = NC * NS
TRASH = NU + NI
ACC_R = NU + NI + 16
C = 64


def _splat_i32(x):
    return jnp.full((L,), 0, jnp.int32) + x


def _sc_body(ue_h, ie_h, p2_h, tu_h, ti_h, tl_h, uu_h, ui_h, ul_h,
             acc_h,
             a_all, b_all, y_all, a_cmp, b_cmp, y_cmp,
             a_idx, b_idx, pa_i, pb_i,
             au_i, bu_i, yu_v,
             ue_r, ie_r, pu_r, pi_r,
             contrib, sidx_t, sidx_u,
             acc_sh,
             sem0, sem1, sem2, sem3):
    cid = lax.axis_index("c")
    sid = lax.axis_index("s")
    gwid = cid * NS + sid
    T = tu_h.shape[0]
    U = uu_h.shape[0]
    TPW = T // NW
    UPW = U // NW
    inv_t = 1.0 / T
    iota = lax.iota(jnp.int32, L)
    zero16 = jnp.zeros((L,), jnp.float32)
    izero16 = jnp.zeros((L,), jnp.int32)

    base = gwid * TPW
    cpa = pltpu.async_copy(tu_h.at[pl.ds(base, TPW)], a_all, sem0)
    cpb = pltpu.async_copy(ti_h.at[pl.ds(base, TPW)], b_all, sem1)
    cpy = pltpu.async_copy(tl_h.at[pl.ds(base, TPW)], y_all, sem2)

    def _zrow(r, _):
        for c4 in range(D // L):
            contrib[r, pl.ds(c4 * L, L)] = zero16
        return 0
    lax.fori_loop(0, 2 * C, _zrow, 0)
    zbase = sid * ((NU + NI) // NS)
    zcps = [
        pltpu.async_copy(contrib,
                         acc_sh.at[pl.ds(zbase + j * 2 * C, 2 * C)], sem3)
        for j in range((NU + NI) // NS // (2 * C))
    ]
    for zc in zcps:
        zc.wait()
    cpa.wait(); cpb.wait(); cpy.wait()
    plsc.subcore_barrier()

    def _zcmp(i, _):
        a_cmp[pl.ds(i * L, L)] = izero16
        b_cmp[pl.ds(i * L, L)] = izero16
        y_cmp[pl.ds(i * L, L)] = zero16
        return 0
    lax.fori_loop(0, TPW // L, _zcmp, 0)

    def _scan(g, n):
        av = a_all[pl.ds(g * L, L)]
        bv = b_all[pl.ds(g * L, L)]
        yv = y_all[pl.ds(g * L, L)]
        act = (av < NU) | (bv < NI)
        ai = jnp.where(act, 1, 0)
        pos = n + plsc.cumsum(ai) - ai
        plsc.store_scatter(a_cmp, [pos], av, mask=act)
        plsc.store_scatter(b_cmp, [pos], bv, mask=act)
        plsc.store_scatter(y_cmp, [pos], yv, mask=act)
        return n + jnp.sum(ai)

    n_act = lax.fori_loop(0, TPW // L, _scan, jnp.int32(0))

    for r in range(TPW // C):
        @pl.when(r * C < n_act)
        def _(r=r):
            for j in range(C // L):
                off = r * C + j * L
                av = a_cmp[pl.ds(off, L)]
                bv = b_cmp[pl.ds(off, L)]
                valid = (off + iota) < n_act
                a_idx[pl.ds(j * L, L)] = av
                b_idx[pl.ds(j * L, L)] = bv
                pa_i[pl.ds(j * L, L)] = jnp.minimum(av, NU - 1)
                pb_i[pl.ds(j * L, L)] = NU + jnp.minimum(bv, NI - 1)
                sidx_t[pl.ds(j * L, L)] = jnp.where(
                    valid & (av < NU), av, TRASH)
                sidx_t[pl.ds(C + j * L, L)] = jnp.where(
                    valid & (bv < NI), NU + bv, TRASH)
            cp0 = pltpu.async_copy(ue_h.at[a_idx], ue_r, sem0)
            cp1 = pltpu.async_copy(ie_h.at[b_idx], ie_r, sem1)
            cp2 = pltpu.async_copy(p2_h.at[pa_i], pu_r, sem2)
            cp3 = pltpu.async_copy(p2_h.at[pb_i], pi_r, sem3)
            cp0.wait(); cp1.wait(); cp2.wait(); cp3.wait()
            for g in range(C // L):
                rvec = iota + g * L
                rvec2 = rvec + C

                def _dots(dd, carry, rvec=rvec):
                    s_a, wu_a, wi_a = carry
                    col = _splat_i32(dd)
                    ue_d = plsc.load_gather(ue_r, [rvec, col])
                    ie_d = plsc.load_gather(ie_r, [rvec, col])
                    pu_d = plsc.load_gather(pu_r, [rvec, col])
                    pi_d = plsc.load_gather(pi_r, [rvec, col])
                    return (s_a + ue_d * ie_d, wu_a + pu_d * ie_d,
                            wi_a + ue_d * pi_d)

                s, wu, wi = lax.fori_loop(0, D, _dots,
                                          (zero16, zero16, zero16), unroll=4)
                av = a_cmp[pl.ds(r * C + g * L, L)]
                bv = b_cmp[pl.ds(r * C + g * L, L)]
                yv = y_cmp[pl.ds(r * C + g * L, L)]
                maf = jnp.where(av < NU, 1.0, 0.0)
                mbf = jnp.where(bv < NI, 1.0, 0.0)
                sg = 1.0 / (1.0 + jnp.exp(-s))
                gp = sg - yv
                hh = sg * (1.0 - sg)
                w = wu * maf + wi * mbf
                c1 = inv_t * hh * w
                cu2 = inv_t * gp * mbf
                ci2 = inv_t * gp * maf

                def _emit(dd, _, rvec=rvec, rvec2=rvec2,
                          c1=c1, cu2=cu2, ci2=ci2):
                    col = _splat_i32(dd)
                    ue_d = plsc.load_gather(ue_r, [rvec, col])
                    ie_d = plsc.load_gather(ie_r, [rvec, col])
                    pu_d = plsc.load_gather(pu_r, [rvec, col])
                    pi_d = plsc.load_gather(pi_r, [rvec, col])
                    plsc.store_scatter(contrib, [rvec, col],
                                       c1 * ie_d + cu2 * pi_d)
                    plsc.store_scatter(contrib, [rvec2, col],
                                       c1 * ue_d + ci2 * pu_d)
                    return 0

                lax.fori_loop(0, D, _emit, 0, unroll=4)
            pltpu.sync_copy(contrib, acc_sh.at[sidx_t], add=True)

    baseu = gwid * UPW
    cpa = pltpu.async_copy(uu_h.at[pl.ds(baseu, UPW)], au_i, sem0)
    cpb = pltpu.async_copy(ui_h.at[pl.ds(baseu, UPW)], bu_i, sem1)
    cpy = pltpu.async_copy(ul_h.at[pl.ds(baseu, UPW)], yu_v, sem2)
    cpa.wait(); cpb.wait(); cpy.wait()
    cp0 = pltpu.async_copy(ue_h.at[au_i], ue_r.at[pl.ds(0, UPW)], sem0)
    cp1 = pltpu.async_copy(ie_h.at[bu_i], ie_r.at[pl.ds(0, UPW)], sem1)
    cp0.wait(); cp1.wait()
    for g in range(UPW // L):
        rvec = iota + g * L
        rvec2 = rvec + UPW

        def _dots_u(dd, s_a, rvec=rvec):
            col = _splat_i32(dd)
            ue_d = plsc.load_gather(ue_r, [rvec, col])
            ie_d = plsc.load_gather(ie_r, [rvec, col])
            return s_a + ue_d * ie_d

        s = lax.fori_loop(0, D, _dots_u, zero16, unroll=4)
        av = au_i[pl.ds(g * L, L)]
        bv = bu_i[pl.ds(g * L, L)]
        yv = yu_v[pl.ds(g * L, L)]
        sg = 1.0 / (1.0 + jnp.exp(-s))
        cg = yv - sg

        def _emit_u(dd, _, rvec=rvec, rvec2=rvec2, cg=cg):
            col = _splat_i32(dd)
            ue_d = plsc.load_gather(ue_r, [rvec, col])
            ie_d = plsc.load_gather(ie_r, [rvec, col])
            plsc.store_scatter(contrib, [rvec, col], cg * ie_d)
            plsc.store_scatter(contrib, [rvec2, col], cg * ue_d)
            return 0

        lax.fori_loop(0, D, _emit_u, 0, unroll=4)
        sidx_u[pl.ds(g * L, L)] = jnp.where(av < NU, av, TRASH)
        sidx_u[pl.ds(UPW + g * L, L)] = jnp.where(bv < NI, NU + bv, TRASH)
    pltpu.sync_copy(contrib.at[pl.ds(0, 2 * UPW)], acc_sh.at[sidx_u],
                    add=True)

    plsc.subcore_barrier()
    rows = (NU + NI) // NS
    pltpu.sync_copy(acc_sh.at[pl.ds(sid * rows, rows)],
                    acc_h.at[cid, pl.ds(sid * rows, rows)])


def _sc_call(user_emb, item_emb, p2, tu, ti, tl, uu, ui, ul):
    T = tu.shape[0]
    U = uu.shape[0]
    mesh = plsc.VectorSubcoreMesh(core_axis_name="c", subcore_axis_name="s")
    UPW = U // NW
    f = pl.kernel(
        _sc_body,
        out_type=jax.ShapeDtypeStruct((NC, NU + NI, D), jnp.float32),
        mesh=mesh,
        compiler_params=pltpu.CompilerParams(
            use_tc_tiling_on_sc=False, needs_layout_passes=False),
        scratch_types=[
            pltpu.VMEM((T // NW,), jnp.int32),
            pltpu.VMEM((T // NW,), jnp.int32),
            pltpu.VMEM((T // NW,), jnp.float32),
            pltpu.VMEM((T // NW,), jnp.int32),
            pltpu.VMEM((T // NW,), jnp.int32),
            pltpu.VMEM((T // NW,), jnp.float32),
            pltpu.VMEM((C,), jnp.int32),
            pltpu.VMEM((C,), jnp.int32),
            pltpu.VMEM((C,), jnp.int32),
            pltpu.VMEM((C,), jnp.int32),
            pltpu.VMEM((UPW,), jnp.int32),
            pltpu.VMEM((UPW,), jnp.int32),
            pltpu.VMEM((UPW,), jnp.float32),
            pltpu.VMEM((C, D), jnp.float32),
            pltpu.VMEM((C, D), jnp.float32),
            pltpu.VMEM((C, D), jnp.float32),
            pltpu.VMEM((C, D), jnp.float32),
            pltpu.VMEM((2 * C, D), jnp.float32),
            pltpu.VMEM((2 * C,), jnp.int32),
            pltpu.VMEM((2 * UPW,), jnp.int32),
            pltpu.VMEM_SHARED((ACC_R, D), jnp.float32),
            pltpu.SemaphoreType.DMA,
            pltpu.SemaphoreType.DMA,
            pltpu.SemaphoreType.DMA,
            pltpu.SemaphoreType.DMA,
        ],
    )
    return f(user_emb, item_emb, p2, tu, ti, tl, uu, ui, ul)


ROWS_BLK = 8192
W = 128
NUW = NU * D // W


def _copy_body(u_ref, i_ref, o_ref):
    gg = pl.program_id(0)

    @pl.when(gg == 0)
    def _():
        o_ref[0, :, :] = u_ref[...]

    @pl.when(gg == 1)
    def _():
        o_ref[0, :, :] = i_ref[...]


def _upd_body(inv_t, b_ref, p_ref, a_ref, o_ref):
    o_ref[...] = b_ref[...] + inv_t * (
        p_ref[0][None] - IF_LR * (a_ref[0] + a_ref[1])[None])


def _tc_call(user_emb, item_emb, p, acc, T):
    n = user_emb.shape[0]
    nw = n * D // W
    u2 = user_emb.reshape(nw, W)
    i2 = item_emb.reshape(nw, W)
    p4 = p.reshape(2, NUW, W)
    a4 = acc.reshape(NC, 2 * NUW, W)
    nblk = (nw + ROWS_BLK - 1) // ROWS_BLK
    big = pl.pallas_call(
        _copy_body,
        grid=(2, nblk),
        in_specs=[
            pl.BlockSpec((ROWS_BLK, W),
                         lambda g, i: (jnp.where(g == 0, i, 0), 0)),
            pl.BlockSpec((ROWS_BLK, W),
                         lambda g, i: (jnp.where(g == 1, i, 0), 0)),
        ],
        out_specs=pl.BlockSpec((1, ROWS_BLK, W), lambda g, i: (g, i, 0)),
        out_shape=jax.ShapeDtypeStruct((2, nw, W), jnp.float32),
    )(u2, i2)
    out = pl.pallas_call(
        functools.partial(_upd_body, 1.0 / T),
        grid=(2,),
        in_specs=[
            pl.BlockSpec((1, NUW, W), lambda g: (g, 0, 0)),
            pl.BlockSpec((1, NUW, W), lambda g: (g, 0, 0)),
            pl.BlockSpec((NC, NUW, W), lambda g: (0, g, 0)),
        ],
        out_specs=pl.BlockSpec((1, NUW, W), lambda g: (g, 0, 0)),
        out_shape=jax.ShapeDtypeStruct((2, nw, W), jnp.float32),
        input_output_aliases={0: 0},
    )(big, p4, a4)
    return out.reshape(2, n, D)


def kernel(user_emb, item_emb, p, train_labels, unlearn_labels, nei_users,
           nei_items, train_users, train_items, unlearn_users, unlearn_items):
    T = train_users.shape[0]
    p2 = p.reshape(NU + NI, D)
    acc = _sc_call(user_emb, item_emb, p2,
                   train_users, train_items, train_labels,
                   unlearn_users, unlearn_items, unlearn_labels)
    return _tc_call(user_emb, item_emb, p, acc, T)


# speedup vs baseline: 1.7353x; 1.7353x over previous
import functools

import jax
import jax.numpy as jnp
from jax import lax
from jax.experimental import pallas as pl
from jax.experimental.pallas import tpu as pltpu
from jax.experimental.pallas import tpu_sc as plsc

NU = 4096
NI = 4096
D = 64
IF_LR = 0.01
NC = 2
NS = 16
L = 16
NW = NC * NS
TRASH = NU + NI
ACC_R = NU + NI + 16
C = 64


def _splat_i32(x):
    return jnp.full((L,), 0, jnp.int32) + x


def _sc_body(ue_h, ie_h, p2_h, tu_h, ti_h, tl_h, uu_h, ui_h, ul_h,
             acc_h,
             a_all, b_all, y_all, a_cmp, b_cmp, y_cmp,
             a_idx, b_idx, pa_i, pb_i,
             au_i, bu_i, yu_v,
             ue_r, ie_r, pu_r, pi_r,
             contrib, sidx_t, sidx_u,
             acc_sh,
             sem0, sem1, sem2, sem3):
    cid = lax.axis_index("c")
    sid = lax.axis_index("s")
    gwid = cid * NS + sid
    T = tu_h.shape[0]
    U = uu_h.shape[0]
    TPW = T // NW
    UPW = U // NW
    inv_t = 1.0 / T
    iota = lax.iota(jnp.int32, L)
    zero16 = jnp.zeros((L,), jnp.float32)
    izero16 = jnp.zeros((L,), jnp.int32)

    base = gwid * TPW
    cpa = pltpu.async_copy(tu_h.at[pl.ds(base, TPW)], a_all, sem0)
    cpb = pltpu.async_copy(ti_h.at[pl.ds(base, TPW)], b_all, sem1)
    cpy = pltpu.async_copy(tl_h.at[pl.ds(base, TPW)], y_all, sem2)

    def _zrow(r, _):
        for c4 in range(D // L):
            contrib[r, pl.ds(c4 * L, L)] = zero16
        return 0
    lax.fori_loop(0, 2 * C, _zrow, 0)
    zbase = sid * ((NU + NI) // NS)
    zcps = [
        pltpu.async_copy(contrib,
                         acc_sh.at[pl.ds(zbase + j * 2 * C, 2 * C)], sem3)
        for j in range((NU + NI) // NS // (2 * C))
    ]
    for zc in zcps:
        zc.wait()
    cpa.wait(); cpb.wait(); cpy.wait()
    plsc.subcore_barrier()

    def _zcmp(i, _):
        a_cmp[pl.ds(i * L, L)] = izero16
        b_cmp[pl.ds(i * L, L)] = izero16
        y_cmp[pl.ds(i * L, L)] = zero16
        return 0
    lax.fori_loop(0, TPW // L, _zcmp, 0)

    def _scan(g, n):
        av = a_all[pl.ds(g * L, L)]
        bv = b_all[pl.ds(g * L, L)]
        yv = y_all[pl.ds(g * L, L)]
        act = (av < NU) | (bv < NI)
        ai = jnp.where(act, 1, 0)
        pos = n + plsc.cumsum(ai) - ai
        plsc.store_scatter(a_cmp, [pos], av, mask=act)
        plsc.store_scatter(b_cmp, [pos], bv, mask=act)
        plsc.store_scatter(y_cmp, [pos], yv, mask=act)
        return n + jnp.sum(ai)

    n_act = lax.fori_loop(0, TPW // L, _scan, jnp.int32(0))

    for r in range(TPW // C):
        @pl.when(r * C < n_act)
        def _(r=r):
            for j in range(C // L):
                off = r * C + j * L
                av = a_cmp[pl.ds(off, L)]
                bv = b_cmp[pl.ds(off, L)]
                valid = (off + iota) < n_act
                a_idx[pl.ds(j * L, L)] = av
                b_idx[pl.ds(j * L, L)] = bv
                pa_i[pl.ds(j * L, L)] = jnp.minimum(av, NU - 1)
                pb_i[pl.ds(j * L, L)] = NU + jnp.minimum(bv, NI - 1)
                sidx_t[pl.ds(j * L, L)] = jnp.where(
                    valid & (av < NU), av, TRASH)
                sidx_t[pl.ds(C + j * L, L)] = jnp.where(
                    valid & (bv < NI), NU + bv, TRASH)
            cp0 = pltpu.async_copy(ue_h.at[a_idx], ue_r, sem0)
            cp1 = pltpu.async_copy(ie_h.at[b_idx], ie_r, sem1)
            cp2 = pltpu.async_copy(p2_h.at[pa_i], pu_r, sem2)
            cp3 = pltpu.async_copy(p2_h.at[pb_i], pi_r, sem3)
            cp0.wait(); cp1.wait(); cp2.wait(); cp3.wait()
            for g in range(C // L):
                rvec = iota + g * L
                rvec2 = rvec + C

                def _dots(dd, carry, rvec=rvec):
                    s_a, wu_a, wi_a = carry
                    col = _splat_i32(dd)
                    ue_d = plsc.load_gather(ue_r, [rvec, col])
                    ie_d = plsc.load_gather(ie_r, [rvec, col])
                    pu_d = plsc.load_gather(pu_r, [rvec, col])
                    pi_d = plsc.load_gather(pi_r, [rvec, col])
                    return (s_a + ue_d * ie_d, wu_a + pu_d * ie_d,
                            wi_a + ue_d * pi_d)

                s, wu, wi = lax.fori_loop(0, D, _dots,
                                          (zero16, zero16, zero16), unroll=4)
                av = a_cmp[pl.ds(r * C + g * L, L)]
                bv = b_cmp[pl.ds(r * C + g * L, L)]
                yv = y_cmp[pl.ds(r * C + g * L, L)]
                maf = jnp.where(av < NU, 1.0, 0.0)
                mbf = jnp.where(bv < NI, 1.0, 0.0)
                sg = 1.0 / (1.0 + jnp.exp(-s))
                gp = sg - yv
                hh = sg * (1.0 - sg)
                w = wu * maf + wi * mbf
                c1 = inv_t * hh * w
                cu2 = inv_t * gp * mbf
                ci2 = inv_t * gp * maf

                def _emit(dd, _, rvec=rvec, rvec2=rvec2,
                          c1=c1, cu2=cu2, ci2=ci2):
                    col = _splat_i32(dd)
                    ue_d = plsc.load_gather(ue_r, [rvec, col])
                    ie_d = plsc.load_gather(ie_r, [rvec, col])
                    pu_d = plsc.load_gather(pu_r, [rvec, col])
                    pi_d = plsc.load_gather(pi_r, [rvec, col])
                    plsc.store_scatter(contrib, [rvec, col],
                                       c1 * ie_d + cu2 * pi_d)
                    plsc.store_scatter(contrib, [rvec2, col],
                                       c1 * ue_d + ci2 * pu_d)
                    return 0

                lax.fori_loop(0, D, _emit, 0, unroll=4)
            pltpu.sync_copy(contrib, acc_sh.at[sidx_t], add=True)

    baseu = gwid * UPW
    cpa = pltpu.async_copy(uu_h.at[pl.ds(baseu, UPW)], au_i, sem0)
    cpb = pltpu.async_copy(ui_h.at[pl.ds(baseu, UPW)], bu_i, sem1)
    cpy = pltpu.async_copy(ul_h.at[pl.ds(baseu, UPW)], yu_v, sem2)
    cpa.wait(); cpb.wait(); cpy.wait()
    cp0 = pltpu.async_copy(ue_h.at[au_i], ue_r.at[pl.ds(0, UPW)], sem0)
    cp1 = pltpu.async_copy(ie_h.at[bu_i], ie_r.at[pl.ds(0, UPW)], sem1)
    cp0.wait(); cp1.wait()
    for g in range(UPW // L):
        rvec = iota + g * L
        rvec2 = rvec + UPW

        def _dots_u(dd, s_a, rvec=rvec):
            col = _splat_i32(dd)
            ue_d = plsc.load_gather(ue_r, [rvec, col])
            ie_d = plsc.load_gather(ie_r, [rvec, col])
            return s_a + ue_d * ie_d

        s = lax.fori_loop(0, D, _dots_u, zero16, unroll=4)
        av = au_i[pl.ds(g * L, L)]
        bv = bu_i[pl.ds(g * L, L)]
        yv = yu_v[pl.ds(g * L, L)]
        sg = 1.0 / (1.0 + jnp.exp(-s))
        cg = yv - sg

        def _emit_u(dd, _, rvec=rvec, rvec2=rvec2, cg=cg):
            col = _splat_i32(dd)
            ue_d = plsc.load_gather(ue_r, [rvec, col])
            ie_d = plsc.load_gather(ie_r, [rvec, col])
            plsc.store_scatter(contrib, [rvec, col], cg * ie_d)
            plsc.store_scatter(contrib, [rvec2, col], cg * ue_d)
            return 0

        lax.fori_loop(0, D, _emit_u, 0, unroll=4)
        sidx_u[pl.ds(g * L, L)] = jnp.where(av < NU, av, TRASH)
        sidx_u[pl.ds(UPW + g * L, L)] = jnp.where(bv < NI, NU + bv, TRASH)
    pltpu.sync_copy(contrib.at[pl.ds(0, 2 * UPW)], acc_sh.at[sidx_u],
                    add=True)

    plsc.subcore_barrier()
    rows = (NU + NI) // NS
    pltpu.sync_copy(acc_sh.at[pl.ds(sid * rows, rows)],
                    acc_h.at[cid, pl.ds(sid * rows, rows)])


def _sc_call(user_emb, item_emb, p2, tu, ti, tl, uu, ui, ul):
    T = tu.shape[0]
    U = uu.shape[0]
    mesh = plsc.VectorSubcoreMesh(core_axis_name="c", subcore_axis_name="s")
    UPW = U // NW
    f = pl.kernel(
        _sc_body,
        out_type=jax.ShapeDtypeStruct((NC, NU + NI, D), jnp.float32),
        mesh=mesh,
        compiler_params=pltpu.CompilerParams(
            use_tc_tiling_on_sc=False, needs_layout_passes=False),
        scratch_types=[
            pltpu.VMEM((T // NW,), jnp.int32),
            pltpu.VMEM((T // NW,), jnp.int32),
            pltpu.VMEM((T // NW,), jnp.float32),
            pltpu.VMEM((T // NW,), jnp.int32),
            pltpu.VMEM((T // NW,), jnp.int32),
            pltpu.VMEM((T // NW,), jnp.float32),
            pltpu.VMEM((C,), jnp.int32),
            pltpu.VMEM((C,), jnp.int32),
            pltpu.VMEM((C,), jnp.int32),
            pltpu.VMEM((C,), jnp.int32),
            pltpu.VMEM((UPW,), jnp.int32),
            pltpu.VMEM((UPW,), jnp.int32),
            pltpu.VMEM((UPW,), jnp.float32),
            pltpu.VMEM((C, D), jnp.float32),
            pltpu.VMEM((C, D), jnp.float32),
            pltpu.VMEM((C, D), jnp.float32),
            pltpu.VMEM((C, D), jnp.float32),
            pltpu.VMEM((2 * C, D), jnp.float32),
            pltpu.VMEM((2 * C,), jnp.int32),
            pltpu.VMEM((2 * UPW,), jnp.int32),
            pltpu.VMEM_SHARED((ACC_R, D), jnp.float32),
            pltpu.SemaphoreType.DMA,
            pltpu.SemaphoreType.DMA,
            pltpu.SemaphoreType.DMA,
            pltpu.SemaphoreType.DMA,
        ],
    )
    return f(user_emb, item_emb, p2, tu, ti, tl, uu, ui, ul)


CBLK = 16384


def _copy_body(u_ref, i_ref, o_ref):
    gg = pl.program_id(0)

    @pl.when(gg == 0)
    def _():
        o_ref[0, :, :] = u_ref[...]

    @pl.when(gg == 1)
    def _():
        o_ref[0, :, :] = i_ref[...]


def _upd_body(inv_t, b_ref, p_ref, a_ref, o_ref):
    o_ref[0] = b_ref[0] + inv_t * (
        p_ref[0] - IF_LR * (a_ref[0, 0] + a_ref[1, 0]))


def _tc_call(user_emb, item_emb, p, acc, T):
    n = user_emb.shape[0]
    uT = user_emb.T
    iT = item_emb.T
    pT = p.reshape(2, NU, D).transpose(0, 2, 1)
    aT = acc.reshape(NC, 2, NU, D).transpose(0, 1, 3, 2)
    nblk = (n + CBLK - 1) // CBLK
    big = pl.pallas_call(
        _copy_body,
        grid=(2, nblk),
        in_specs=[
            pl.BlockSpec((D, CBLK),
                         lambda g, i: (0, jnp.where(g == 0, i, 0))),
            pl.BlockSpec((D, CBLK),
                         lambda g, i: (0, jnp.where(g == 1, i, 0))),
        ],
        out_specs=pl.BlockSpec((1, D, CBLK), lambda g, i: (g, 0, i)),
        out_shape=jax.ShapeDtypeStruct((2, D, n), jnp.float32),
    )(uT, iT)
    out = pl.pallas_call(
        functools.partial(_upd_body, 1.0 / T),
        grid=(2,),
        in_specs=[
            pl.BlockSpec((1, D, NU), lambda g: (g, 0, 0)),
            pl.BlockSpec((1, D, NU), lambda g: (g, 0, 0)),
            pl.BlockSpec((NC, 1, D, NU), lambda g: (0, g, 0, 0)),
        ],
        out_specs=pl.BlockSpec((1, D, NU), lambda g: (g, 0, 0)),
        out_shape=jax.ShapeDtypeStruct((2, D, n), jnp.float32),
        input_output_aliases={0: 0},
    )(big, pT, aT)
    return out.transpose(0, 2, 1)


def kernel(user_emb, item_emb, p, train_labels, unlearn_labels, nei_users,
           nei_items, train_users, train_items, unlearn_users, unlearn_items):
    T = train_users.shape[0]
    n = user_emb.shape[0]
    u_sc = lax.optimization_barrier(user_emb.reshape(-1)).reshape(n, D)
    i_sc = lax.optimization_barrier(item_emb.reshape(-1)).reshape(n, D)
    p2 = p.reshape(NU + NI, D)
    acc = _sc_call(u_sc, i_sc, p2,
                   train_users, train_items, train_labels,
                   unlearn_users, unlearn_items, unlearn_labels)
    return _tc_call(user_emb, item_emb, p, acc, T)
